# baseline (speedup 1.0000x reference)
import jax
import jax.numpy as jnp
from jax import lax
from jax.experimental import pallas as pl
from jax.experimental.pallas import tpu as pltpu
from jax.experimental.pallas import tpu_sc as plsc

_MAX_OBJ = 100
_IOU_THR = 0.45
_SCORE_THR = 0.25
_BIG_I32 = 2**30
_L = 16
_OUTW = 112
_K_TC = 10
_SPB = 5


def _prep_tc_kernel(x_ref, s_ref, cls_ref, x1_ref, y1_ref, x2_ref, y2_ref,
                    ar_ref):
    blk = x_ref[0]
    cx = blk[0:1, :]
    cy = blk[1:2, :]
    w = blk[2:3, :]
    h = blk[3:4, :]
    x1 = cx - w / 2
    y1 = cy - h / 2
    x2 = cx + w / 2
    y2 = cy + h / 2

    scores = blk[4:, :]
    m = jnp.max(scores, axis=0, keepdims=True)
    iota_c = lax.broadcasted_iota(jnp.int32, scores.shape, 0)
    cls = jnp.min(jnp.where(scores == m, iota_c, _BIG_I32),
                  axis=0, keepdims=True)

    s_ref[0] = jnp.where(m > _SCORE_THR, m, -1.0)
    cls_ref[0] = cls
    x1_ref[0] = x1
    y1_ref[0] = y1
    x2_ref[0] = x2
    y2_ref[0] = y2
    ar_ref[0] = (x2 - x1) * (y2 - y1)


def _tc_nms_kernel(s_ref, cls_ref, x1_ref, y1_ref, x2_ref, y2_ref,
                   nd_ref, sc_ref, cl_ref, ix_ref,
                   b0_ref, b1_ref, b2_ref, b3_ref):
    S0 = s_ref[...]
    CLS = cls_ref[...]
    X1 = x1_ref[...]
    Y1 = y1_ref[...]
    X2 = x2_ref[...]
    Y2 = y2_ref[...]
    AREA = (X2 - X1) * (Y2 - Y1)
    K, N = S0.shape
    iota_n = lax.broadcasted_iota(jnp.int32, (K, N), 1)
    col = lax.broadcasted_iota(jnp.int32, (K, _MAX_OBJ), 1)

    def body(t, carry):
        S, nd, asc, acl, aix, ab0, ab1, ab2, ab3 = carry
        best = jnp.max(S, axis=1, keepdims=True)
        idx = jnp.min(jnp.where(S == best, iota_n, _BIG_I32),
                      axis=1, keepdims=True)
        onehot = iota_n == idx
        bx1 = jnp.sum(jnp.where(onehot, X1, 0.0), axis=1, keepdims=True)
        by1 = jnp.sum(jnp.where(onehot, Y1, 0.0), axis=1, keepdims=True)
        bx2 = jnp.sum(jnp.where(onehot, X2, 0.0), axis=1, keepdims=True)
        by2 = jnp.sum(jnp.where(onehot, Y2, 0.0), axis=1, keepdims=True)
        bcl = jnp.sum(jnp.where(onehot, CLS, 0), axis=1, keepdims=True)

        ix1 = jnp.maximum(bx1, X1)
        iy1 = jnp.maximum(by1, Y1)
        ix2 = jnp.minimum(bx2, X2)
        iy2 = jnp.minimum(by2, Y2)
        inter = jnp.clip(ix2 - ix1, 0.0) * jnp.clip(iy2 - iy1, 0.0)
        area1 = (bx2 - bx1) * (by2 - by1)
        iou = inter / (area1 + AREA - inter + 1e-9)

        S = jnp.where(iou > _IOU_THR, -1.0, S)
        S = jnp.where(onehot, -1.0, S)

        keep = best > _SCORE_THR
        sel = col == t
        asc = jnp.where(sel, jnp.where(keep, best, 0.0), asc)
        acl = jnp.where(sel, jnp.where(keep, bcl, -1), acl)
        aix = jnp.where(sel, idx, aix)
        ab0 = jnp.where(sel, jnp.where(keep, (bx1 + bx2) * 0.5, 0.0), ab0)
        ab1 = jnp.where(sel, jnp.where(keep, (by1 + by2) * 0.5, 0.0), ab1)
        ab2 = jnp.where(sel, jnp.where(keep, bx2 - bx1, 0.0), ab2)
        ab3 = jnp.where(sel, jnp.where(keep, by2 - by1, 0.0), ab3)
        nd = nd + keep.astype(jnp.int32)
        return (S, nd, asc, acl, aix, ab0, ab1, ab2, ab3)

    init = (S0,
            jnp.zeros((K, 1), jnp.int32),
            jnp.zeros((K, _MAX_OBJ), jnp.float32),
            jnp.zeros((K, _MAX_OBJ), jnp.int32),
            jnp.zeros((K, _MAX_OBJ), jnp.int32),
            jnp.zeros((K, _MAX_OBJ), jnp.float32),
            jnp.zeros((K, _MAX_OBJ), jnp.float32),
            jnp.zeros((K, _MAX_OBJ), jnp.float32),
            jnp.zeros((K, _MAX_OBJ), jnp.float32))
    (_, nd, asc, acl, aix, ab0, ab1, ab2, ab3) = lax.fori_loop(
        0, _MAX_OBJ, body, init)

    nd_ref[...] = nd
    sc_ref[...] = asc
    cl_ref[...] = acl
    ix_ref[...] = aix
    b0_ref[...] = ab0
    b1_ref[...] = ab1
    b2_ref[...] = ab2
    b3_ref[...] = ab3


def _sc_nms_body(s_h, cls_h, x1_h, y1_h, x2_h, y2_h, ar_h,
                 nd_h, sc_h, cl_h, ix_h, b0_h, b1_h, b2_h, b3_h,
                 s_v, cls_v, x1_v, y1_v, x2_v, y2_v, ar_v,
                 ix_v, kp_v, scv_v, c0_v, c1_v, c2_v, c3_v, cg_v, nd_v,
                 msg_v, grp_v, shr_v):
    cid = lax.axis_index("c")
    sid = lax.axis_index("s")
    g = sid // _SPB
    j = sid % _SPB
    ngrp = 16 // _SPB
    active = g < ngrp
    brow = cid * ngrp + g
    bs = ngrp * 2
    n = s_h.shape[0] // bs
    qpad = s_v.shape[0]
    qtail = n - (_SPB - 1) * qpad
    nch = qpad // _L
    base = pl.multiple_of(j * qpad, _L)
    off = pl.multiple_of(jnp.where(active, brow * n + base, 0), _L)

    @pl.when(active & (j < _SPB - 1))
    def _load_full():
        pltpu.sync_copy(s_h.at[pl.ds(off, qpad)], s_v)
        pltpu.sync_copy(cls_h.at[pl.ds(off, qpad)], cls_v)
        pltpu.sync_copy(x1_h.at[pl.ds(off, qpad)], x1_v)
        pltpu.sync_copy(y1_h.at[pl.ds(off, qpad)], y1_v)
        pltpu.sync_copy(x2_h.at[pl.ds(off, qpad)], x2_v)
        pltpu.sync_copy(y2_h.at[pl.ds(off, qpad)], y2_v)
        pltpu.sync_copy(ar_h.at[pl.ds(off, qpad)], ar_v)

    @pl.when(active & (j == _SPB - 1))
    def _load_tail():
        dst = pl.ds(0, qtail)
        pltpu.sync_copy(s_h.at[pl.ds(off, qtail)], s_v.at[dst])
        pltpu.sync_copy(cls_h.at[pl.ds(off, qtail)], cls_v.at[dst])
        pltpu.sync_copy(x1_h.at[pl.ds(off, qtail)], x1_v.at[dst])
        pltpu.sync_copy(y1_h.at[pl.ds(off, qtail)], y1_v.at[dst])
        pltpu.sync_copy(x2_h.at[pl.ds(off, qtail)], x2_v.at[dst])
        pltpu.sync_copy(y2_h.at[pl.ds(off, qtail)], y2_v.at[dst])
        pltpu.sync_copy(ar_h.at[pl.ds(off, qtail)], ar_v.at[dst])
        for kk in range(qtail, qpad, _L):
            ds = pl.ds(kk, _L)
            s_v[ds] = jnp.full((_L,), -1.0, jnp.float32)
            cls_v[ds] = jnp.zeros((_L,), jnp.int32)
            x1_v[ds] = jnp.zeros((_L,), jnp.float32)
            y1_v[ds] = jnp.zeros((_L,), jnp.float32)
            x2_v[ds] = jnp.zeros((_L,), jnp.float32)
            y2_v[ds] = jnp.zeros((_L,), jnp.float32)
            ar_v[ds] = jnp.zeros((_L,), jnp.float32)

    @pl.when(jnp.logical_not(active))
    def _fill_idle():
        def fill(k, c):
            s_v[pl.ds(k * _L, _L)] = jnp.full((_L,), -1.0, jnp.float32)
            x1_v[pl.ds(k * _L, _L)] = jnp.zeros((_L,), jnp.float32)
            y1_v[pl.ds(k * _L, _L)] = jnp.zeros((_L,), jnp.float32)
            x2_v[pl.ds(k * _L, _L)] = jnp.zeros((_L,), jnp.float32)
            y2_v[pl.ds(k * _L, _L)] = jnp.zeros((_L,), jnp.float32)
            ar_v[pl.ds(k * _L, _L)] = jnp.zeros((_L,), jnp.float32)
            cls_v[pl.ds(k * _L, _L)] = jnp.zeros((_L,), jnp.int32)
            return c
        lax.fori_loop(0, nch, fill, 0)

    lanes = lax.iota(jnp.int32, _L)
    lane0 = lanes == 0
    zf = jnp.zeros((_L,), jnp.float32)
    zi = jnp.zeros((_L,), jnp.int32)
    for k in range(_OUTW // _L):
        ds = pl.ds(k * _L, _L)
        ix_v[ds] = zi
        kp_v[ds] = zf
        scv_v[ds] = zf
        c0_v[ds] = zf
        c1_v[ds] = zf
        c2_v[ds] = zf
        c3_v[ds] = zf
        cg_v[ds] = zi

    def splat(v, e):
        return v.at[jnp.full((_L,), e, jnp.int32)].get(
            mode="promise_in_bounds")

    def argreduce(m, mi):
        for sh in (8, 4, 2, 1):
            pidx = lanes ^ sh
            m2 = m.at[pidx].get(mode="promise_in_bounds")
            mi2 = mi.at[pidx].get(mode="promise_in_bounds")
            swap = (m2 > m) | ((m2 == m) & (mi2 < mi))
            m = jnp.where(swap, m2, m)
            mi = jnp.where(swap, mi2, mi)
        return m, mi

    @plsc.parallel_loop(0, nch, 1, unroll=8,
                        carry=(jnp.full((_L,), -2.0, jnp.float32),
                               jnp.zeros((_L,), jnp.int32)))
    def pro(k, carry):
        m, mi = carry
        v = s_v[pl.ds(k * _L, _L)]
        gg = lanes + (base + k * _L)
        cmp = v > m
        return (jnp.where(cmp, v, m), jnp.where(cmp, gg, mi))

    m, mi = pro

    def iter_body(t, carry):
        m, mi, nd = carry
        lb, li = argreduce(m, mi)
        liv = li - base
        bx1 = plsc.load_gather(x1_v, [liv])
        by1 = plsc.load_gather(y1_v, [liv])
        bx2 = plsc.load_gather(x2_v, [liv])
        by2 = plsc.load_gather(y2_v, [liv])
        bcl = plsc.load_gather(cls_v, [liv])
        msg = jnp.where(lanes == 0, lb,
              jnp.where(lanes == 1, li.astype(jnp.float32),
              jnp.where(lanes == 2, bx1,
              jnp.where(lanes == 3, by1,
              jnp.where(lanes == 4, bx2,
              jnp.where(lanes == 5, by2,
                        bcl.astype(jnp.float32)))))))
        msg_v[...] = msg
        pltpu.sync_copy(msg_v, shr_v.at[pl.ds(pl.multiple_of(sid * _L, _L),
                                              _L)])
        plsc.subcore_barrier()
        g_safe = jnp.where(active, g, 0)
        pltpu.sync_copy(
            shr_v.at[pl.ds(pl.multiple_of(g_safe * (_SPB * _L), _SPB * _L),
                           _SPB * _L)],
            grp_v)
        plsc.subcore_barrier()

        win = grp_v[pl.ds(0, _L)]
        wb = splat(win, 0)
        wi = splat(win, 1)
        for jj in range(1, _SPB):
            r = grp_v[pl.ds(jj * _L, _L)]
            rb = splat(r, 0)
            ri = splat(r, 1)
            better = (rb > wb) | ((rb == wb) & (ri < wi))
            win = jnp.where(better, r, win)
            wb = jnp.where(better, rb, wb)
            wi = jnp.where(better, ri, wi)
        bestv = wb
        iv = wi.astype(jnp.int32)
        bx1 = splat(win, 2)
        by1 = splat(win, 3)
        bx2 = splat(win, 4)
        by2 = splat(win, 5)
        bcli = splat(win, 6).astype(jnp.int32)
        area1 = (bx2 - bx1) * (by2 - by1)

        keepv = bestv > _SCORE_THR
        kfv = jnp.where(keepv, 1.0, 0.0)

        @pl.when(active & (j == 0))
        def _record():
            tv = jnp.full((_L,), t, jnp.int32)
            plsc.store_scatter(ix_v, [tv], iv, mask=lane0)
            plsc.store_scatter(kp_v, [tv], kfv, mask=lane0)
            plsc.store_scatter(scv_v, [tv], bestv * kfv, mask=lane0)
            plsc.store_scatter(c0_v, [tv], (bx1 + bx2) * 0.5 * kfv,
                               mask=lane0)
            plsc.store_scatter(c1_v, [tv], (by1 + by2) * 0.5 * kfv,
                               mask=lane0)
            plsc.store_scatter(c2_v, [tv], (bx2 - bx1) * kfv, mask=lane0)
            plsc.store_scatter(c3_v, [tv], (by2 - by1) * kfv, mask=lane0)
            plsc.store_scatter(cg_v, [tv],
                               jnp.where(keepv, bcli, -1), mask=lane0)
        nd = nd + jnp.where(keepv, 1, 0)

        @plsc.parallel_loop(0, nch, 1, unroll=8,
                            carry=(jnp.full((_L,), -2.0, jnp.float32),
                                   jnp.zeros((_L,), jnp.int32)))
        def sweep(k, carry):
            m, mi = carry
            ds = pl.ds(k * _L, _L)
            sk = s_v[ds]
            ax1 = x1_v[ds]
            ay1 = y1_v[ds]
            ax2 = x2_v[ds]
            ay2 = y2_v[ds]
            ix1 = jnp.maximum(bx1, ax1)
            iy1 = jnp.maximum(by1, ay1)
            ix2 = jnp.minimum(bx2, ax2)
            iy2 = jnp.minimum(by2, ay2)
            inter = (jnp.maximum(ix2 - ix1, 0.0)
                     * jnp.maximum(iy2 - iy1, 0.0))
            area2 = ar_v[ds]
            iou = inter / (area1 + area2 - inter + 1e-9)
            gg = lanes + (base + k * _L)
            snew = jnp.where(iou > _IOU_THR, -1.0, sk)
            snew = jnp.where(gg == iv, -1.0, snew)
            s_v[ds] = snew
            cmp = snew > m
            return (jnp.where(cmp, snew, m), jnp.where(cmp, gg, mi))

        m, mi = sweep
        return (m, mi, nd)

    m, mi, nd = lax.fori_loop(
        0, _MAX_OBJ, iter_body,
        (m, mi, jnp.zeros((_L,), jnp.int32)))

    @pl.when(active & (j == 0))
    def _writeout():
        nd_v[...] = nd
        ondo = pl.multiple_of(brow * _L, _L)
        outo = pl.multiple_of(brow * _OUTW, _L)
        pltpu.sync_copy(nd_v, nd_h.at[pl.ds(ondo, _L)])
        pltpu.sync_copy(scv_v, sc_h.at[pl.ds(outo, _OUTW)])
        pltpu.sync_copy(cg_v, cl_h.at[pl.ds(outo, _OUTW)])
        pltpu.sync_copy(ix_v, ix_h.at[pl.ds(outo, _OUTW)])
        pltpu.sync_copy(c0_v, b0_h.at[pl.ds(outo, _OUTW)])
        pltpu.sync_copy(c1_v, b1_h.at[pl.ds(outo, _OUTW)])
        pltpu.sync_copy(c2_v, b2_h.at[pl.ds(outo, _OUTW)])
        pltpu.sync_copy(c3_v, b3_h.at[pl.ds(outo, _OUTW)])


def kernel(x):
    B, C, N = x.shape
    f32 = jnp.float32
    i32 = jnp.int32

    prep_shapes = (
        jax.ShapeDtypeStruct((B, 1, N), f32),
        jax.ShapeDtypeStruct((B, 1, N), i32),
        jax.ShapeDtypeStruct((B, 1, N), f32),
        jax.ShapeDtypeStruct((B, 1, N), f32),
        jax.ShapeDtypeStruct((B, 1, N), f32),
        jax.ShapeDtypeStruct((B, 1, N), f32),
        jax.ShapeDtypeStruct((B, 1, N), f32),
    )
    prep_out = pl.pallas_call(
        _prep_tc_kernel,
        grid=(B,),
        in_specs=[pl.BlockSpec((1, C, N), lambda b: (b, 0, 0))],
        out_specs=tuple(pl.BlockSpec((1, 1, N), lambda b: (b, 0, 0))
                        for _ in prep_shapes),
        out_shape=prep_shapes,
        compiler_params=pltpu.CompilerParams(
            dimension_semantics=("parallel",)),
    )(x)
    s, cls, x1, y1, x2, y2, area = (jnp.reshape(a, (B, N)) for a in prep_out)

    K = _K_TC
    tc_shapes = (
        jax.ShapeDtypeStruct((K, 1), i32),
        jax.ShapeDtypeStruct((K, _MAX_OBJ), f32),
        jax.ShapeDtypeStruct((K, _MAX_OBJ), i32),
        jax.ShapeDtypeStruct((K, _MAX_OBJ), i32),
        jax.ShapeDtypeStruct((K, _MAX_OBJ), f32),
        jax.ShapeDtypeStruct((K, _MAX_OBJ), f32),
        jax.ShapeDtypeStruct((K, _MAX_OBJ), f32),
        jax.ShapeDtypeStruct((K, _MAX_OBJ), f32),
    )
    tnd, tsc, tcl, tix, tb0, tb1, tb2, tb3 = pl.pallas_call(
        _tc_nms_kernel,
        out_shape=tc_shapes,
    )(s[:K], cls[:K], x1[:K], y1[:K], x2[:K], y2[:K])

    BS = B - K
    qpad = ((N + _SPB - 1) // _SPB + _L - 1) // _L * _L
    sc_types = (
        jax.ShapeDtypeStruct((BS * _L,), i32),
        jax.ShapeDtypeStruct((BS * _OUTW,), f32),
        jax.ShapeDtypeStruct((BS * _OUTW,), i32),
        jax.ShapeDtypeStruct((BS * _OUTW,), i32),
        jax.ShapeDtypeStruct((BS * _OUTW,), f32),
        jax.ShapeDtypeStruct((BS * _OUTW,), f32),
        jax.ShapeDtypeStruct((BS * _OUTW,), f32),
        jax.ShapeDtypeStruct((BS * _OUTW,), f32),
    )
    scratch = [
        pltpu.VMEM((qpad,), f32),
        pltpu.VMEM((qpad,), i32),
        pltpu.VMEM((qpad,), f32),
        pltpu.VMEM((qpad,), f32),
        pltpu.VMEM((qpad,), f32),
        pltpu.VMEM((qpad,), f32),
        pltpu.VMEM((qpad,), f32),
        pltpu.VMEM((_OUTW,), i32),
        pltpu.VMEM((_OUTW,), f32),
        pltpu.VMEM((_OUTW,), f32),
        pltpu.VMEM((_OUTW,), f32),
        pltpu.VMEM((_OUTW,), f32),
        pltpu.VMEM((_OUTW,), f32),
        pltpu.VMEM((_OUTW,), f32),
        pltpu.VMEM((_OUTW,), i32),
        pltpu.VMEM((_L,), i32),
        pltpu.VMEM((_L,), f32),
        pltpu.VMEM((_SPB * _L,), f32),
        pltpu.VMEM_SHARED((16 * _L,), f32),
    ]
    nms = pl.kernel(
        _sc_nms_body,
        out_type=sc_types,
        mesh=plsc.VectorSubcoreMesh(core_axis_name="c",
                                    subcore_axis_name="s"),
        scratch_types=scratch,
        compiler_params=pltpu.CompilerParams(needs_layout_passes=False),
    )
    flat = lambda a: jnp.reshape(a[K:], (-1,))
    snd, ssc, scl, six, sb0, sb1, sb2, sb3 = nms(
        flat(s), flat(cls), flat(x1), flat(y1), flat(x2), flat(y2),
        flat(area))
    snd = jnp.reshape(snd, (BS, _L))
    ssc = jnp.reshape(ssc, (BS, _OUTW))
    scl = jnp.reshape(scl, (BS, _OUTW))
    six = jnp.reshape(six, (BS, _OUTW))
    sb0 = jnp.reshape(sb0, (BS, _OUTW))
    sb1 = jnp.reshape(sb1, (BS, _OUTW))
    sb2 = jnp.reshape(sb2, (BS, _OUTW))
    sb3 = jnp.reshape(sb3, (BS, _OUTW))

    nd = jnp.concatenate([tnd, snd[:, :1]], axis=0)
    asc = jnp.concatenate([tsc, ssc[:, :_MAX_OBJ]], axis=0)
    acl = jnp.concatenate([tcl, scl[:, :_MAX_OBJ]], axis=0)
    aix = jnp.concatenate([tix, six[:, :_MAX_OBJ]], axis=0)
    ab0 = jnp.concatenate([tb0, sb0[:, :_MAX_OBJ]], axis=0)
    ab1 = jnp.concatenate([tb1, sb1[:, :_MAX_OBJ]], axis=0)
    ab2 = jnp.concatenate([tb2, sb2[:, :_MAX_OBJ]], axis=0)
    ab3 = jnp.concatenate([tb3, sb3[:, :_MAX_OBJ]], axis=0)
    det_boxes = jnp.stack([ab0, ab1, ab2, ab3], axis=-1)
    return (nd, det_boxes, asc, acl, aix)


# speedup vs baseline: 1.3581x; 1.3581x over previous
import jax
import jax.numpy as jnp
from jax import lax
from jax.experimental import pallas as pl
from jax.experimental.pallas import tpu as pltpu
from jax.experimental.pallas import tpu_sc as plsc

_MAX_OBJ = 100
_IOU_THR = 0.45
_SCORE_THR = 0.25
_BIG_I32 = 2**30
_L = 16
_OUTW = 112
_K_TC = 8
_SPB = 4


def _prep_tc_kernel(x_ref, s_ref, cls_ref, x1_ref, y1_ref, x2_ref, y2_ref,
                    ar_ref):
    blk = x_ref[0]
    cx = blk[0:1, :]
    cy = blk[1:2, :]
    w = blk[2:3, :]
    h = blk[3:4, :]
    x1 = cx - w / 2
    y1 = cy - h / 2
    x2 = cx + w / 2
    y2 = cy + h / 2

    scores = blk[4:, :]
    m = jnp.max(scores, axis=0, keepdims=True)
    iota_c = lax.broadcasted_iota(jnp.int32, scores.shape, 0)
    cls = jnp.min(jnp.where(scores == m, iota_c, _BIG_I32),
                  axis=0, keepdims=True)

    s_ref[0] = jnp.where(m > _SCORE_THR, m, -1.0)
    cls_ref[0] = cls
    x1_ref[0] = x1
    y1_ref[0] = y1
    x2_ref[0] = x2
    y2_ref[0] = y2
    ar_ref[0] = (x2 - x1) * (y2 - y1)


def _tc_nms_kernel(s_ref, cls_ref, x1_ref, y1_ref, x2_ref, y2_ref,
                   nd_ref, sc_ref, cl_ref, ix_ref,
                   b0_ref, b1_ref, b2_ref, b3_ref):
    S0 = s_ref[...]
    CLS = cls_ref[...]
    X1 = x1_ref[...]
    Y1 = y1_ref[...]
    X2 = x2_ref[...]
    Y2 = y2_ref[...]
    AREA = (X2 - X1) * (Y2 - Y1)
    K, N = S0.shape
    iota_n = lax.broadcasted_iota(jnp.int32, (K, N), 1)
    col = lax.broadcasted_iota(jnp.int32, (K, _MAX_OBJ), 1)

    def body(t, carry):
        S, nd, asc, acl, aix, ab0, ab1, ab2, ab3 = carry
        best = jnp.max(S, axis=1, keepdims=True)
        idx = jnp.min(jnp.where(S == best, iota_n, _BIG_I32),
                      axis=1, keepdims=True)
        onehot = iota_n == idx
        bx1 = jnp.sum(jnp.where(onehot, X1, 0.0), axis=1, keepdims=True)
        by1 = jnp.sum(jnp.where(onehot, Y1, 0.0), axis=1, keepdims=True)
        bx2 = jnp.sum(jnp.where(onehot, X2, 0.0), axis=1, keepdims=True)
        by2 = jnp.sum(jnp.where(onehot, Y2, 0.0), axis=1, keepdims=True)
        bcl = jnp.sum(jnp.where(onehot, CLS, 0), axis=1, keepdims=True)

        ix1 = jnp.maximum(bx1, X1)
        iy1 = jnp.maximum(by1, Y1)
        ix2 = jnp.minimum(bx2, X2)
        iy2 = jnp.minimum(by2, Y2)
        inter = jnp.clip(ix2 - ix1, 0.0) * jnp.clip(iy2 - iy1, 0.0)
        area1 = (bx2 - bx1) * (by2 - by1)
        iou = inter / (area1 + AREA - inter + 1e-9)

        S = jnp.where(iou > _IOU_THR, -1.0, S)
        S = jnp.where(onehot, -1.0, S)

        keep = best > _SCORE_THR
        sel = col == t
        asc = jnp.where(sel, jnp.where(keep, best, 0.0), asc)
        acl = jnp.where(sel, jnp.where(keep, bcl, -1), acl)
        aix = jnp.where(sel, idx, aix)
        ab0 = jnp.where(sel, jnp.where(keep, (bx1 + bx2) * 0.5, 0.0), ab0)
        ab1 = jnp.where(sel, jnp.where(keep, (by1 + by2) * 0.5, 0.0), ab1)
        ab2 = jnp.where(sel, jnp.where(keep, bx2 - bx1, 0.0), ab2)
        ab3 = jnp.where(sel, jnp.where(keep, by2 - by1, 0.0), ab3)
        nd = nd + keep.astype(jnp.int32)
        return (S, nd, asc, acl, aix, ab0, ab1, ab2, ab3)

    init = (S0,
            jnp.zeros((K, 1), jnp.int32),
            jnp.zeros((K, _MAX_OBJ), jnp.float32),
            jnp.zeros((K, _MAX_OBJ), jnp.int32),
            jnp.zeros((K, _MAX_OBJ), jnp.int32),
            jnp.zeros((K, _MAX_OBJ), jnp.float32),
            jnp.zeros((K, _MAX_OBJ), jnp.float32),
            jnp.zeros((K, _MAX_OBJ), jnp.float32),
            jnp.zeros((K, _MAX_OBJ), jnp.float32))
    (_, nd, asc, acl, aix, ab0, ab1, ab2, ab3) = lax.fori_loop(
        0, _MAX_OBJ, body, init)

    nd_ref[...] = nd
    sc_ref[...] = asc
    cl_ref[...] = acl
    ix_ref[...] = aix
    b0_ref[...] = ab0
    b1_ref[...] = ab1
    b2_ref[...] = ab2
    b3_ref[...] = ab3


def _sc_nms_body(s_h, cls_h, x1_h, y1_h, x2_h, y2_h, ar_h,
                 nd_h, sc_h, cl_h, ix_h, b0_h, b1_h, b2_h, b3_h,
                 s_v, cls_v, x1_v, y1_v, x2_v, y2_v, ar_v,
                 ix_v, kp_v, scv_v, c0_v, c1_v, c2_v, c3_v, cg_v, nd_v,
                 msg_v, grp_v, shr_v):
    cid = lax.axis_index("c")
    sid = lax.axis_index("s")
    g = sid // _SPB
    j = sid % _SPB
    ngrp = 16 // _SPB
    active = g < ngrp
    brow = cid * ngrp + g
    bs = ngrp * 2
    n = s_h.shape[0] // bs
    qpad = s_v.shape[0]
    qtail = n - (_SPB - 1) * qpad
    nch = qpad // _L
    base = pl.multiple_of(j * qpad, _L)
    off = pl.multiple_of(jnp.where(active, brow * n + base, 0), _L)

    @pl.when(active & (j < _SPB - 1))
    def _load_full():
        pltpu.sync_copy(s_h.at[pl.ds(off, qpad)], s_v)
        pltpu.sync_copy(cls_h.at[pl.ds(off, qpad)], cls_v)
        pltpu.sync_copy(x1_h.at[pl.ds(off, qpad)], x1_v)
        pltpu.sync_copy(y1_h.at[pl.ds(off, qpad)], y1_v)
        pltpu.sync_copy(x2_h.at[pl.ds(off, qpad)], x2_v)
        pltpu.sync_copy(y2_h.at[pl.ds(off, qpad)], y2_v)
        pltpu.sync_copy(ar_h.at[pl.ds(off, qpad)], ar_v)

    @pl.when(active & (j == _SPB - 1))
    def _load_tail():
        dst = pl.ds(0, qtail)
        pltpu.sync_copy(s_h.at[pl.ds(off, qtail)], s_v.at[dst])
        pltpu.sync_copy(cls_h.at[pl.ds(off, qtail)], cls_v.at[dst])
        pltpu.sync_copy(x1_h.at[pl.ds(off, qtail)], x1_v.at[dst])
        pltpu.sync_copy(y1_h.at[pl.ds(off, qtail)], y1_v.at[dst])
        pltpu.sync_copy(x2_h.at[pl.ds(off, qtail)], x2_v.at[dst])
        pltpu.sync_copy(y2_h.at[pl.ds(off, qtail)], y2_v.at[dst])
        pltpu.sync_copy(ar_h.at[pl.ds(off, qtail)], ar_v.at[dst])
        for kk in range(qtail, qpad, _L):
            ds = pl.ds(kk, _L)
            s_v[ds] = jnp.full((_L,), -1.0, jnp.float32)
            cls_v[ds] = jnp.zeros((_L,), jnp.int32)
            x1_v[ds] = jnp.zeros((_L,), jnp.float32)
            y1_v[ds] = jnp.zeros((_L,), jnp.float32)
            x2_v[ds] = jnp.zeros((_L,), jnp.float32)
            y2_v[ds] = jnp.zeros((_L,), jnp.float32)
            ar_v[ds] = jnp.zeros((_L,), jnp.float32)

    @pl.when(jnp.logical_not(active))
    def _fill_idle():
        def fill(k, c):
            s_v[pl.ds(k * _L, _L)] = jnp.full((_L,), -1.0, jnp.float32)
            x1_v[pl.ds(k * _L, _L)] = jnp.zeros((_L,), jnp.float32)
            y1_v[pl.ds(k * _L, _L)] = jnp.zeros((_L,), jnp.float32)
            x2_v[pl.ds(k * _L, _L)] = jnp.zeros((_L,), jnp.float32)
            y2_v[pl.ds(k * _L, _L)] = jnp.zeros((_L,), jnp.float32)
            ar_v[pl.ds(k * _L, _L)] = jnp.zeros((_L,), jnp.float32)
            cls_v[pl.ds(k * _L, _L)] = jnp.zeros((_L,), jnp.int32)
            return c
        lax.fori_loop(0, nch, fill, 0)

    lanes = lax.iota(jnp.int32, _L)
    lane0 = lanes == 0
    zf = jnp.zeros((_L,), jnp.float32)
    zi = jnp.zeros((_L,), jnp.int32)
    for k in range(_OUTW // _L):
        ds = pl.ds(k * _L, _L)
        ix_v[ds] = zi
        kp_v[ds] = zf
        scv_v[ds] = zf
        c0_v[ds] = zf
        c1_v[ds] = zf
        c2_v[ds] = zf
        c3_v[ds] = zf
        cg_v[ds] = zi

    def splat(v, e):
        return v.at[jnp.full((_L,), e, jnp.int32)].get(
            mode="promise_in_bounds")

    def argreduce(m, mi):
        for sh in (8, 4, 2, 1):
            pidx = lanes ^ sh
            m2 = m.at[pidx].get(mode="promise_in_bounds")
            mi2 = mi.at[pidx].get(mode="promise_in_bounds")
            swap = (m2 > m) | ((m2 == m) & (mi2 < mi))
            m = jnp.where(swap, m2, m)
            mi = jnp.where(swap, mi2, mi)
        return m, mi

    @plsc.parallel_loop(0, nch, 1, unroll=8,
                        carry=(jnp.full((_L,), -2.0, jnp.float32),
                               jnp.zeros((_L,), jnp.int32)))
    def pro(k, carry):
        m, mi = carry
        v = s_v[pl.ds(k * _L, _L)]
        gg = lanes + (base + k * _L)
        cmp = v > m
        return (jnp.where(cmp, v, m), jnp.where(cmp, gg, mi))

    m, mi = pro

    def iter_body(t, carry):
        m, mi, nd = carry
        lb, li = argreduce(m, mi)
        liv = li - base
        bx1 = plsc.load_gather(x1_v, [liv])
        by1 = plsc.load_gather(y1_v, [liv])
        bx2 = plsc.load_gather(x2_v, [liv])
        by2 = plsc.load_gather(y2_v, [liv])
        bcl = plsc.load_gather(cls_v, [liv])
        msg = jnp.where(lanes == 0, lb,
              jnp.where(lanes == 1, li.astype(jnp.float32),
              jnp.where(lanes == 2, bx1,
              jnp.where(lanes == 3, by1,
              jnp.where(lanes == 4, bx2,
              jnp.where(lanes == 5, by2,
                        bcl.astype(jnp.float32)))))))
        msg_v[...] = msg
        pltpu.sync_copy(msg_v, shr_v.at[pl.ds(pl.multiple_of(sid * _L, _L),
                                              _L)])
        plsc.subcore_barrier()
        g_safe = jnp.where(active, g, 0)
        pltpu.sync_copy(
            shr_v.at[pl.ds(pl.multiple_of(g_safe * (_SPB * _L), _SPB * _L),
                           _SPB * _L)],
            grp_v)
        plsc.subcore_barrier()

        win = grp_v[pl.ds(0, _L)]
        wb = splat(win, 0)
        wi = splat(win, 1)
        for jj in range(1, _SPB):
            r = grp_v[pl.ds(jj * _L, _L)]
            rb = splat(r, 0)
            ri = splat(r, 1)
            better = (rb > wb) | ((rb == wb) & (ri < wi))
            win = jnp.where(better, r, win)
            wb = jnp.where(better, rb, wb)
            wi = jnp.where(better, ri, wi)
        bestv = wb
        iv = wi.astype(jnp.int32)
        bx1 = splat(win, 2)
        by1 = splat(win, 3)
        bx2 = splat(win, 4)
        by2 = splat(win, 5)
        bcli = splat(win, 6).astype(jnp.int32)
        area1 = (bx2 - bx1) * (by2 - by1)

        keepv = bestv > _SCORE_THR
        kfv = jnp.where(keepv, 1.0, 0.0)

        @pl.when(active & (j == 0))
        def _record():
            tv = jnp.full((_L,), t, jnp.int32)
            plsc.store_scatter(ix_v, [tv], iv, mask=lane0)
            plsc.store_scatter(kp_v, [tv], kfv, mask=lane0)
            plsc.store_scatter(scv_v, [tv], bestv * kfv, mask=lane0)
            plsc.store_scatter(c0_v, [tv], (bx1 + bx2) * 0.5 * kfv,
                               mask=lane0)
            plsc.store_scatter(c1_v, [tv], (by1 + by2) * 0.5 * kfv,
                               mask=lane0)
            plsc.store_scatter(c2_v, [tv], (bx2 - bx1) * kfv, mask=lane0)
            plsc.store_scatter(c3_v, [tv], (by2 - by1) * kfv, mask=lane0)
            plsc.store_scatter(cg_v, [tv],
                               jnp.where(keepv, bcli, -1), mask=lane0)
        nd = nd + jnp.where(keepv, 1, 0)

        @plsc.parallel_loop(0, nch, 1, unroll=8,
                            carry=(jnp.full((_L,), -2.0, jnp.float32),
                                   jnp.zeros((_L,), jnp.int32)))
        def sweep(k, carry):
            m, mi = carry
            ds = pl.ds(k * _L, _L)
            sk = s_v[ds]
            ax1 = x1_v[ds]
            ay1 = y1_v[ds]
            ax2 = x2_v[ds]
            ay2 = y2_v[ds]
            ix1 = jnp.maximum(bx1, ax1)
            iy1 = jnp.maximum(by1, ay1)
            ix2 = jnp.minimum(bx2, ax2)
            iy2 = jnp.minimum(by2, ay2)
            inter = (jnp.maximum(ix2 - ix1, 0.0)
                     * jnp.maximum(iy2 - iy1, 0.0))
            area2 = ar_v[ds]
            iou = inter / (area1 + area2 - inter + 1e-9)
            gg = lanes + (base + k * _L)
            snew = jnp.where(iou > _IOU_THR, -1.0, sk)
            snew = jnp.where(gg == iv, -1.0, snew)
            s_v[ds] = snew
            cmp = snew > m
            return (jnp.where(cmp, snew, m), jnp.where(cmp, gg, mi))

        m, mi = sweep
        return (m, mi, nd)

    m, mi, nd = lax.fori_loop(
        0, _MAX_OBJ, iter_body,
        (m, mi, jnp.zeros((_L,), jnp.int32)))

    @pl.when(active & (j == 0))
    def _writeout():
        nd_v[...] = nd
        ondo = pl.multiple_of(brow * _L, _L)
        outo = pl.multiple_of(brow * _OUTW, _L)
        pltpu.sync_copy(nd_v, nd_h.at[pl.ds(ondo, _L)])
        pltpu.sync_copy(scv_v, sc_h.at[pl.ds(outo, _OUTW)])
        pltpu.sync_copy(cg_v, cl_h.at[pl.ds(outo, _OUTW)])
        pltpu.sync_copy(ix_v, ix_h.at[pl.ds(outo, _OUTW)])
        pltpu.sync_copy(c0_v, b0_h.at[pl.ds(outo, _OUTW)])
        pltpu.sync_copy(c1_v, b1_h.at[pl.ds(outo, _OUTW)])
        pltpu.sync_copy(c2_v, b2_h.at[pl.ds(outo, _OUTW)])
        pltpu.sync_copy(c3_v, b3_h.at[pl.ds(outo, _OUTW)])


def kernel(x):
    B, C, N = x.shape
    f32 = jnp.float32
    i32 = jnp.int32

    prep_shapes = (
        jax.ShapeDtypeStruct((B, 1, N), f32),
        jax.ShapeDtypeStruct((B, 1, N), i32),
        jax.ShapeDtypeStruct((B, 1, N), f32),
        jax.ShapeDtypeStruct((B, 1, N), f32),
        jax.ShapeDtypeStruct((B, 1, N), f32),
        jax.ShapeDtypeStruct((B, 1, N), f32),
        jax.ShapeDtypeStruct((B, 1, N), f32),
    )
    prep_out = pl.pallas_call(
        _prep_tc_kernel,
        grid=(B,),
        in_specs=[pl.BlockSpec((1, C, N), lambda b: (b, 0, 0))],
        out_specs=tuple(pl.BlockSpec((1, 1, N), lambda b: (b, 0, 0))
                        for _ in prep_shapes),
        out_shape=prep_shapes,
        compiler_params=pltpu.CompilerParams(
            dimension_semantics=("parallel",)),
    )(x)
    s, cls, x1, y1, x2, y2, area = (jnp.reshape(a, (B, N)) for a in prep_out)

    K = _K_TC
    tc_shapes = (
        jax.ShapeDtypeStruct((K, 1), i32),
        jax.ShapeDtypeStruct((K, _MAX_OBJ), f32),
        jax.ShapeDtypeStruct((K, _MAX_OBJ), i32),
        jax.ShapeDtypeStruct((K, _MAX_OBJ), i32),
        jax.ShapeDtypeStruct((K, _MAX_OBJ), f32),
        jax.ShapeDtypeStruct((K, _MAX_OBJ), f32),
        jax.ShapeDtypeStruct((K, _MAX_OBJ), f32),
        jax.ShapeDtypeStruct((K, _MAX_OBJ), f32),
    )
    tnd, tsc, tcl, tix, tb0, tb1, tb2, tb3 = pl.pallas_call(
        _tc_nms_kernel,
        out_shape=tc_shapes,
    )(s[:K], cls[:K], x1[:K], y1[:K], x2[:K], y2[:K])

    BS = B - K
    qpad = ((N + _SPB - 1) // _SPB + _L - 1) // _L * _L
    sc_types = (
        jax.ShapeDtypeStruct((BS * _L,), i32),
        jax.ShapeDtypeStruct((BS * _OUTW,), f32),
        jax.ShapeDtypeStruct((BS * _OUTW,), i32),
        jax.ShapeDtypeStruct((BS * _OUTW,), i32),
        jax.ShapeDtypeStruct((BS * _OUTW,), f32),
        jax.ShapeDtypeStruct((BS * _OUTW,), f32),
        jax.ShapeDtypeStruct((BS * _OUTW,), f32),
        jax.ShapeDtypeStruct((BS * _OUTW,), f32),
    )
    scratch = [
        pltpu.VMEM((qpad,), f32),
        pltpu.VMEM((qpad,), i32),
        pltpu.VMEM((qpad,), f32),
        pltpu.VMEM((qpad,), f32),
        pltpu.VMEM((qpad,), f32),
        pltpu.VMEM((qpad,), f32),
        pltpu.VMEM((qpad,), f32),
        pltpu.VMEM((_OUTW,), i32),
        pltpu.VMEM((_OUTW,), f32),
        pltpu.VMEM((_OUTW,), f32),
        pltpu.VMEM((_OUTW,), f32),
        pltpu.VMEM((_OUTW,), f32),
        pltpu.VMEM((_OUTW,), f32),
        pltpu.VMEM((_OUTW,), f32),
        pltpu.VMEM((_OUTW,), i32),
        pltpu.VMEM((_L,), i32),
        pltpu.VMEM((_L,), f32),
        pltpu.VMEM((_SPB * _L,), f32),
        pltpu.VMEM_SHARED((16 * _L,), f32),
    ]
    nms = pl.kernel(
        _sc_nms_body,
        out_type=sc_types,
        mesh=plsc.VectorSubcoreMesh(core_axis_name="c",
                                    subcore_axis_name="s"),
        scratch_types=scratch,
        compiler_params=pltpu.CompilerParams(needs_layout_passes=False),
    )
    flat = lambda a: jnp.reshape(a[K:], (-1,))
    snd, ssc, scl, six, sb0, sb1, sb2, sb3 = nms(
        flat(s), flat(cls), flat(x1), flat(y1), flat(x2), flat(y2),
        flat(area))
    snd = jnp.reshape(snd, (BS, _L))
    ssc = jnp.reshape(ssc, (BS, _OUTW))
    scl = jnp.reshape(scl, (BS, _OUTW))
    six = jnp.reshape(six, (BS, _OUTW))
    sb0 = jnp.reshape(sb0, (BS, _OUTW))
    sb1 = jnp.reshape(sb1, (BS, _OUTW))
    sb2 = jnp.reshape(sb2, (BS, _OUTW))
    sb3 = jnp.reshape(sb3, (BS, _OUTW))

    nd = jnp.concatenate([tnd, snd[:, :1]], axis=0)
    asc = jnp.concatenate([tsc, ssc[:, :_MAX_OBJ]], axis=0)
    acl = jnp.concatenate([tcl, scl[:, :_MAX_OBJ]], axis=0)
    aix = jnp.concatenate([tix, six[:, :_MAX_OBJ]], axis=0)
    ab0 = jnp.concatenate([tb0, sb0[:, :_MAX_OBJ]], axis=0)
    ab1 = jnp.concatenate([tb1, sb1[:, :_MAX_OBJ]], axis=0)
    ab2 = jnp.concatenate([tb2, sb2[:, :_MAX_OBJ]], axis=0)
    ab3 = jnp.concatenate([tb3, sb3[:, :_MAX_OBJ]], axis=0)
    det_boxes = jnp.stack([ab0, ab1, ab2, ab3], axis=-1)
    return (nd, det_boxes, asc, acl, aix)


# speedup vs baseline: 1.3851x; 1.0199x over previous
import jax
import jax.numpy as jnp
from jax import lax
from jax.experimental import pallas as pl
from jax.experimental.pallas import tpu as pltpu
from jax.experimental.pallas import tpu_sc as plsc

_MAX_OBJ = 100
_IOU_THR = 0.45
_SCORE_THR = 0.25
_BIG_I32 = 2**30
_L = 16
_OUTW = 112
_K_TC = 8
_SPB = 4


def _prep_tc_kernel(x_ref, s_ref, cls_ref, x1_ref, y1_ref, x2_ref, y2_ref,
                    ar_ref):
    blk = x_ref[0]
    cx = blk[0:1, :]
    cy = blk[1:2, :]
    w = blk[2:3, :]
    h = blk[3:4, :]
    x1 = cx - w / 2
    y1 = cy - h / 2
    x2 = cx + w / 2
    y2 = cy + h / 2

    scores = blk[4:, :]
    m = jnp.max(scores, axis=0, keepdims=True)
    iota_c = lax.broadcasted_iota(jnp.int32, scores.shape, 0)
    cls = jnp.min(jnp.where(scores == m, iota_c, _BIG_I32),
                  axis=0, keepdims=True)

    s_ref[0] = jnp.where(m > _SCORE_THR, m, -1.0)
    cls_ref[0] = cls
    x1_ref[0] = x1
    y1_ref[0] = y1
    x2_ref[0] = x2
    y2_ref[0] = y2
    ar_ref[0] = (x2 - x1) * (y2 - y1)


def _tc_nms_kernel(s_ref, cls_ref, x1_ref, y1_ref, x2_ref, y2_ref,
                   nd_ref, sc_ref, cl_ref, ix_ref,
                   b0_ref, b1_ref, b2_ref, b3_ref):
    S0 = s_ref[...]
    CLS = cls_ref[...]
    X1 = x1_ref[...]
    Y1 = y1_ref[...]
    X2 = x2_ref[...]
    Y2 = y2_ref[...]
    AREA = (X2 - X1) * (Y2 - Y1)
    K, N = S0.shape
    iota_n = lax.broadcasted_iota(jnp.int32, (K, N), 1)
    col = lax.broadcasted_iota(jnp.int32, (K, _MAX_OBJ), 1)

    def body(t, carry):
        S, nd, asc, acl, aix, ab0, ab1, ab2, ab3 = carry
        best = jnp.max(S, axis=1, keepdims=True)
        idx = jnp.min(jnp.where(S == best, iota_n, _BIG_I32),
                      axis=1, keepdims=True)
        onehot = iota_n == idx
        bx1 = jnp.sum(jnp.where(onehot, X1, 0.0), axis=1, keepdims=True)
        by1 = jnp.sum(jnp.where(onehot, Y1, 0.0), axis=1, keepdims=True)
        bx2 = jnp.sum(jnp.where(onehot, X2, 0.0), axis=1, keepdims=True)
        by2 = jnp.sum(jnp.where(onehot, Y2, 0.0), axis=1, keepdims=True)
        bcl = jnp.sum(jnp.where(onehot, CLS, 0), axis=1, keepdims=True)

        ix1 = jnp.maximum(bx1, X1)
        iy1 = jnp.maximum(by1, Y1)
        ix2 = jnp.minimum(bx2, X2)
        iy2 = jnp.minimum(by2, Y2)
        inter = jnp.clip(ix2 - ix1, 0.0) * jnp.clip(iy2 - iy1, 0.0)
        area1 = (bx2 - bx1) * (by2 - by1)
        iou = inter / (area1 + AREA - inter + 1e-9)

        S = jnp.where(iou > _IOU_THR, -1.0, S)
        S = jnp.where(onehot, -1.0, S)

        keep = best > _SCORE_THR
        sel = col == t
        asc = jnp.where(sel, jnp.where(keep, best, 0.0), asc)
        acl = jnp.where(sel, jnp.where(keep, bcl, -1), acl)
        aix = jnp.where(sel, idx, aix)
        ab0 = jnp.where(sel, jnp.where(keep, (bx1 + bx2) * 0.5, 0.0), ab0)
        ab1 = jnp.where(sel, jnp.where(keep, (by1 + by2) * 0.5, 0.0), ab1)
        ab2 = jnp.where(sel, jnp.where(keep, bx2 - bx1, 0.0), ab2)
        ab3 = jnp.where(sel, jnp.where(keep, by2 - by1, 0.0), ab3)
        nd = nd + keep.astype(jnp.int32)
        return (S, nd, asc, acl, aix, ab0, ab1, ab2, ab3)

    init = (S0,
            jnp.zeros((K, 1), jnp.int32),
            jnp.zeros((K, _MAX_OBJ), jnp.float32),
            jnp.zeros((K, _MAX_OBJ), jnp.int32),
            jnp.zeros((K, _MAX_OBJ), jnp.int32),
            jnp.zeros((K, _MAX_OBJ), jnp.float32),
            jnp.zeros((K, _MAX_OBJ), jnp.float32),
            jnp.zeros((K, _MAX_OBJ), jnp.float32),
            jnp.zeros((K, _MAX_OBJ), jnp.float32))
    (_, nd, asc, acl, aix, ab0, ab1, ab2, ab3) = lax.fori_loop(
        0, _MAX_OBJ, body, init)

    nd_ref[...] = nd
    sc_ref[...] = asc
    cl_ref[...] = acl
    ix_ref[...] = aix
    b0_ref[...] = ab0
    b1_ref[...] = ab1
    b2_ref[...] = ab2
    b3_ref[...] = ab3


def _sc_nms_body(s_h, cls_h, x1_h, y1_h, x2_h, y2_h, ar_h,
                 nd_h, sc_h, cl_h, ix_h, b0_h, b1_h, b2_h, b3_h,
                 s_v, cls_v, x1_v, y1_v, x2_v, y2_v, ar_v,
                 ix_v, kp_v, scv_v, c0_v, c1_v, c2_v, c3_v, cg_v, nd_v,
                 msg_v, grp_v, shr_v):
    cid = lax.axis_index("c")
    sid = lax.axis_index("s")
    g = sid // _SPB
    j = sid % _SPB
    ngrp = 16 // _SPB
    active = g < ngrp
    brow = cid * ngrp + g
    bs = ngrp * 2
    n = s_h.shape[0] // bs
    qpad = s_v.shape[0]
    qtail = n - (_SPB - 1) * qpad
    nch = qpad // _L
    base = pl.multiple_of(j * qpad, _L)
    off = pl.multiple_of(jnp.where(active, brow * n + base, 0), _L)

    @pl.when(active & (j < _SPB - 1))
    def _load_full():
        pltpu.sync_copy(s_h.at[pl.ds(off, qpad)], s_v)
        pltpu.sync_copy(cls_h.at[pl.ds(off, qpad)], cls_v)
        pltpu.sync_copy(x1_h.at[pl.ds(off, qpad)], x1_v)
        pltpu.sync_copy(y1_h.at[pl.ds(off, qpad)], y1_v)
        pltpu.sync_copy(x2_h.at[pl.ds(off, qpad)], x2_v)
        pltpu.sync_copy(y2_h.at[pl.ds(off, qpad)], y2_v)
        pltpu.sync_copy(ar_h.at[pl.ds(off, qpad)], ar_v)

    @pl.when(active & (j == _SPB - 1))
    def _load_tail():
        dst = pl.ds(0, qtail)
        pltpu.sync_copy(s_h.at[pl.ds(off, qtail)], s_v.at[dst])
        pltpu.sync_copy(cls_h.at[pl.ds(off, qtail)], cls_v.at[dst])
        pltpu.sync_copy(x1_h.at[pl.ds(off, qtail)], x1_v.at[dst])
        pltpu.sync_copy(y1_h.at[pl.ds(off, qtail)], y1_v.at[dst])
        pltpu.sync_copy(x2_h.at[pl.ds(off, qtail)], x2_v.at[dst])
        pltpu.sync_copy(y2_h.at[pl.ds(off, qtail)], y2_v.at[dst])
        pltpu.sync_copy(ar_h.at[pl.ds(off, qtail)], ar_v.at[dst])
        for kk in range(qtail, qpad, _L):
            ds = pl.ds(kk, _L)
            s_v[ds] = jnp.full((_L,), -1.0, jnp.float32)
            cls_v[ds] = jnp.zeros((_L,), jnp.int32)
            x1_v[ds] = jnp.zeros((_L,), jnp.float32)
            y1_v[ds] = jnp.zeros((_L,), jnp.float32)
            x2_v[ds] = jnp.zeros((_L,), jnp.float32)
            y2_v[ds] = jnp.zeros((_L,), jnp.float32)
            ar_v[ds] = jnp.zeros((_L,), jnp.float32)

    @pl.when(jnp.logical_not(active))
    def _fill_idle():
        def fill(k, c):
            s_v[pl.ds(k * _L, _L)] = jnp.full((_L,), -1.0, jnp.float32)
            x1_v[pl.ds(k * _L, _L)] = jnp.zeros((_L,), jnp.float32)
            y1_v[pl.ds(k * _L, _L)] = jnp.zeros((_L,), jnp.float32)
            x2_v[pl.ds(k * _L, _L)] = jnp.zeros((_L,), jnp.float32)
            y2_v[pl.ds(k * _L, _L)] = jnp.zeros((_L,), jnp.float32)
            ar_v[pl.ds(k * _L, _L)] = jnp.zeros((_L,), jnp.float32)
            cls_v[pl.ds(k * _L, _L)] = jnp.zeros((_L,), jnp.int32)
            return c
        lax.fori_loop(0, nch, fill, 0)

    lanes = lax.iota(jnp.int32, _L)
    lane0 = lanes == 0
    zf = jnp.zeros((_L,), jnp.float32)
    zi = jnp.zeros((_L,), jnp.int32)
    for k in range(_OUTW // _L):
        ds = pl.ds(k * _L, _L)
        ix_v[ds] = zi
        kp_v[ds] = zf
        scv_v[ds] = zf
        c0_v[ds] = zf
        c1_v[ds] = zf
        c2_v[ds] = zf
        c3_v[ds] = zf
        cg_v[ds] = zi

    def splat(v, e):
        return v.at[jnp.full((_L,), e, jnp.int32)].get(
            mode="promise_in_bounds")

    def argreduce(m, mi):
        for sh in (8, 4, 2, 1):
            pidx = lanes ^ sh
            m2 = m.at[pidx].get(mode="promise_in_bounds")
            mi2 = mi.at[pidx].get(mode="promise_in_bounds")
            swap = (m2 > m) | ((m2 == m) & (mi2 < mi))
            m = jnp.where(swap, m2, m)
            mi = jnp.where(swap, mi2, mi)
        return m, mi

    @plsc.parallel_loop(0, nch, 1, unroll=8,
                        carry=(jnp.full((_L,), -2.0, jnp.float32),
                               jnp.zeros((_L,), jnp.int32)))
    def pro(k, carry):
        m, mi = carry
        v = s_v[pl.ds(k * _L, _L)]
        gg = lanes + (base + k * _L)
        cmp = v > m
        return (jnp.where(cmp, v, m), jnp.where(cmp, gg, mi))

    m, mi = pro

    def iter_body(t, carry):
        m, mi, nd = carry
        lb, li = argreduce(m, mi)
        liv = li - base
        bx1 = plsc.load_gather(x1_v, [liv])
        by1 = plsc.load_gather(y1_v, [liv])
        bx2 = plsc.load_gather(x2_v, [liv])
        by2 = plsc.load_gather(y2_v, [liv])
        bcl = plsc.load_gather(cls_v, [liv])
        msg = jnp.where(lanes == 0, lb,
              jnp.where(lanes == 1, li.astype(jnp.float32),
              jnp.where(lanes == 2, bx1,
              jnp.where(lanes == 3, by1,
              jnp.where(lanes == 4, bx2,
              jnp.where(lanes == 5, by2,
                        bcl.astype(jnp.float32)))))))
        msg_v[...] = msg
        pltpu.sync_copy(msg_v, shr_v.at[pl.ds(pl.multiple_of(sid * _L, _L),
                                              _L)])
        plsc.subcore_barrier()
        g_safe = jnp.where(active, g, 0)
        pltpu.sync_copy(
            shr_v.at[pl.ds(pl.multiple_of(g_safe * (_SPB * _L), _SPB * _L),
                           _SPB * _L)],
            grp_v)
        plsc.subcore_barrier()

        win = grp_v[pl.ds(0, _L)]
        wb = splat(win, 0)
        wi = splat(win, 1)
        for jj in range(1, _SPB):
            r = grp_v[pl.ds(jj * _L, _L)]
            rb = splat(r, 0)
            ri = splat(r, 1)
            better = (rb > wb) | ((rb == wb) & (ri < wi))
            win = jnp.where(better, r, win)
            wb = jnp.where(better, rb, wb)
            wi = jnp.where(better, ri, wi)
        bestv = wb
        iv = wi.astype(jnp.int32)
        bx1 = splat(win, 2)
        by1 = splat(win, 3)
        bx2 = splat(win, 4)
        by2 = splat(win, 5)
        bcli = splat(win, 6).astype(jnp.int32)
        area1 = (bx2 - bx1) * (by2 - by1)

        keepv = bestv > _SCORE_THR
        kfv = jnp.where(keepv, 1.0, 0.0)

        livc = jnp.clip(iv - base, 0, qpad - 1)
        owm = lane0 & (iv >= base) & (iv < base + qpad)
        plsc.store_scatter(s_v, [livc], jnp.full((_L,), -1.0, jnp.float32),
                           mask=owm)

        @pl.when(active & (j == 0))
        def _record():
            tv = jnp.full((_L,), t, jnp.int32)
            plsc.store_scatter(ix_v, [tv], iv, mask=lane0)
            plsc.store_scatter(kp_v, [tv], kfv, mask=lane0)
            plsc.store_scatter(scv_v, [tv], bestv * kfv, mask=lane0)
            plsc.store_scatter(c0_v, [tv], (bx1 + bx2) * 0.5 * kfv,
                               mask=lane0)
            plsc.store_scatter(c1_v, [tv], (by1 + by2) * 0.5 * kfv,
                               mask=lane0)
            plsc.store_scatter(c2_v, [tv], (bx2 - bx1) * kfv, mask=lane0)
            plsc.store_scatter(c3_v, [tv], (by2 - by1) * kfv, mask=lane0)
            plsc.store_scatter(cg_v, [tv],
                               jnp.where(keepv, bcli, -1), mask=lane0)
        nd = nd + jnp.where(keepv, 1, 0)

        @plsc.parallel_loop(0, nch, 1, unroll=8,
                            carry=(jnp.full((_L,), -2.0, jnp.float32),
                                   jnp.zeros((_L,), jnp.int32)))
        def sweep(k, carry):
            m, mi = carry
            ds = pl.ds(k * _L, _L)
            sk = s_v[ds]
            ax1 = x1_v[ds]
            ay1 = y1_v[ds]
            ax2 = x2_v[ds]
            ay2 = y2_v[ds]
            ix1 = jnp.maximum(bx1, ax1)
            iy1 = jnp.maximum(by1, ay1)
            ix2 = jnp.minimum(bx2, ax2)
            iy2 = jnp.minimum(by2, ay2)
            inter = (jnp.maximum(ix2 - ix1, 0.0)
                     * jnp.maximum(iy2 - iy1, 0.0))
            area2 = ar_v[ds]
            iou = inter / (area1 + area2 - inter + 1e-9)
            gg = lanes + (base + k * _L)
            snew = jnp.where(iou > _IOU_THR, -1.0, sk)
            s_v[ds] = snew
            cmp = snew > m
            return (jnp.where(cmp, snew, m), jnp.where(cmp, gg, mi))

        m, mi = sweep
        return (m, mi, nd)

    m, mi, nd = lax.fori_loop(
        0, _MAX_OBJ, iter_body,
        (m, mi, jnp.zeros((_L,), jnp.int32)))

    @pl.when(active & (j == 0))
    def _writeout():
        nd_v[...] = nd
        ondo = pl.multiple_of(brow * _L, _L)
        outo = pl.multiple_of(brow * _OUTW, _L)
        pltpu.sync_copy(nd_v, nd_h.at[pl.ds(ondo, _L)])
        pltpu.sync_copy(scv_v, sc_h.at[pl.ds(outo, _OUTW)])
        pltpu.sync_copy(cg_v, cl_h.at[pl.ds(outo, _OUTW)])
        pltpu.sync_copy(ix_v, ix_h.at[pl.ds(outo, _OUTW)])
        pltpu.sync_copy(c0_v, b0_h.at[pl.ds(outo, _OUTW)])
        pltpu.sync_copy(c1_v, b1_h.at[pl.ds(outo, _OUTW)])
        pltpu.sync_copy(c2_v, b2_h.at[pl.ds(outo, _OUTW)])
        pltpu.sync_copy(c3_v, b3_h.at[pl.ds(outo, _OUTW)])


def kernel(x):
    B, C, N = x.shape
    f32 = jnp.float32
    i32 = jnp.int32

    prep_shapes = (
        jax.ShapeDtypeStruct((B, 1, N), f32),
        jax.ShapeDtypeStruct((B, 1, N), i32),
        jax.ShapeDtypeStruct((B, 1, N), f32),
        jax.ShapeDtypeStruct((B, 1, N), f32),
        jax.ShapeDtypeStruct((B, 1, N), f32),
        jax.ShapeDtypeStruct((B, 1, N), f32),
        jax.ShapeDtypeStruct((B, 1, N), f32),
    )
    prep_out = pl.pallas_call(
        _prep_tc_kernel,
        grid=(B,),
        in_specs=[pl.BlockSpec((1, C, N), lambda b: (b, 0, 0))],
        out_specs=tuple(pl.BlockSpec((1, 1, N), lambda b: (b, 0, 0))
                        for _ in prep_shapes),
        out_shape=prep_shapes,
        compiler_params=pltpu.CompilerParams(
            dimension_semantics=("parallel",)),
    )(x)
    s, cls, x1, y1, x2, y2, area = (jnp.reshape(a, (B, N)) for a in prep_out)

    K = _K_TC
    tc_shapes = (
        jax.ShapeDtypeStruct((K, 1), i32),
        jax.ShapeDtypeStruct((K, _MAX_OBJ), f32),
        jax.ShapeDtypeStruct((K, _MAX_OBJ), i32),
        jax.ShapeDtypeStruct((K, _MAX_OBJ), i32),
        jax.ShapeDtypeStruct((K, _MAX_OBJ), f32),
        jax.ShapeDtypeStruct((K, _MAX_OBJ), f32),
        jax.ShapeDtypeStruct((K, _MAX_OBJ), f32),
        jax.ShapeDtypeStruct((K, _MAX_OBJ), f32),
    )
    tnd, tsc, tcl, tix, tb0, tb1, tb2, tb3 = pl.pallas_call(
        _tc_nms_kernel,
        out_shape=tc_shapes,
    )(s[:K], cls[:K], x1[:K], y1[:K], x2[:K], y2[:K])

    BS = B - K
    qpad = ((N + _SPB - 1) // _SPB + _L - 1) // _L * _L
    sc_types = (
        jax.ShapeDtypeStruct((BS * _L,), i32),
        jax.ShapeDtypeStruct((BS * _OUTW,), f32),
        jax.ShapeDtypeStruct((BS * _OUTW,), i32),
        jax.ShapeDtypeStruct((BS * _OUTW,), i32),
        jax.ShapeDtypeStruct((BS * _OUTW,), f32),
        jax.ShapeDtypeStruct((BS * _OUTW,), f32),
        jax.ShapeDtypeStruct((BS * _OUTW,), f32),
        jax.ShapeDtypeStruct((BS * _OUTW,), f32),
    )
    scratch = [
        pltpu.VMEM((qpad,), f32),
        pltpu.VMEM((qpad,), i32),
        pltpu.VMEM((qpad,), f32),
        pltpu.VMEM((qpad,), f32),
        pltpu.VMEM((qpad,), f32),
        pltpu.VMEM((qpad,), f32),
        pltpu.VMEM((qpad,), f32),
        pltpu.VMEM((_OUTW,), i32),
        pltpu.VMEM((_OUTW,), f32),
        pltpu.VMEM((_OUTW,), f32),
        pltpu.VMEM((_OUTW,), f32),
        pltpu.VMEM((_OUTW,), f32),
        pltpu.VMEM((_OUTW,), f32),
        pltpu.VMEM((_OUTW,), f32),
        pltpu.VMEM((_OUTW,), i32),
        pltpu.VMEM((_L,), i32),
        pltpu.VMEM((_L,), f32),
        pltpu.VMEM((_SPB * _L,), f32),
        pltpu.VMEM_SHARED((16 * _L,), f32),
    ]
    nms = pl.kernel(
        _sc_nms_body,
        out_type=sc_types,
        mesh=plsc.VectorSubcoreMesh(core_axis_name="c",
                                    subcore_axis_name="s"),
        scratch_types=scratch,
        compiler_params=pltpu.CompilerParams(needs_layout_passes=False),
    )
    flat = lambda a: jnp.reshape(a[K:], (-1,))
    snd, ssc, scl, six, sb0, sb1, sb2, sb3 = nms(
        flat(s), flat(cls), flat(x1), flat(y1), flat(x2), flat(y2),
        flat(area))
    snd = jnp.reshape(snd, (BS, _L))
    ssc = jnp.reshape(ssc, (BS, _OUTW))
    scl = jnp.reshape(scl, (BS, _OUTW))
    six = jnp.reshape(six, (BS, _OUTW))
    sb0 = jnp.reshape(sb0, (BS, _OUTW))
    sb1 = jnp.reshape(sb1, (BS, _OUTW))
    sb2 = jnp.reshape(sb2, (BS, _OUTW))
    sb3 = jnp.reshape(sb3, (BS, _OUTW))

    nd = jnp.concatenate([tnd, snd[:, :1]], axis=0)
    asc = jnp.concatenate([tsc, ssc[:, :_MAX_OBJ]], axis=0)
    acl = jnp.concatenate([tcl, scl[:, :_MAX_OBJ]], axis=0)
    aix = jnp.concatenate([tix, six[:, :_MAX_OBJ]], axis=0)
    ab0 = jnp.concatenate([tb0, sb0[:, :_MAX_OBJ]], axis=0)
    ab1 = jnp.concatenate([tb1, sb1[:, :_MAX_OBJ]], axis=0)
    ab2 = jnp.concatenate([tb2, sb2[:, :_MAX_OBJ]], axis=0)
    ab3 = jnp.concatenate([tb3, sb3[:, :_MAX_OBJ]], axis=0)
    det_boxes = jnp.stack([ab0, ab1, ab2, ab3], axis=-1)
    return (nd, det_boxes, asc, acl, aix)
